# baseline (speedup 1.0000x reference)
import dataclasses
import functools

import jax
import jax.numpy as jnp
from jax import lax
from jax.experimental import pallas as pl
from jax.experimental.pallas import tpu as pltpu
from jax.experimental.pallas import tpu_sc as plsc

N = 10000
E = 320000
C_FEAT = 128
WIDEC = 512
NG = 16

NW = 32
CH = 128
NCHUNK = E // CH
ITERS = -(-NCHUNK // NW)
NPAD = 10240
STRIPE = NPAD // 16

_MESH = plsc.VectorSubcoreMesh(core_axis_name="c", subcore_axis_name="s")
_HI = jax.lax.Precision.HIGHEST

_CP = pltpu.CompilerParams()
if "needs_layout_passes" in pltpu.CompilerParams.__dataclass_fields__:
    _CP = dataclasses.replace(_CP, needs_layout_passes=False)


def _dot(a, b):
    return jnp.dot(a, b, preferred_element_type=jnp.float32, precision=_HI)


@functools.partial(
    pl.kernel,
    out_type=jax.ShapeDtypeStruct((E,), jnp.float32),
    mesh=_MESH,
    compiler_params=_CP,
    scratch_types=[
        pltpu.VMEM((N,), jnp.float32),
        pltpu.VMEM((N,), jnp.float32),
        pltpu.VMEM((N,), jnp.float32),
        pltpu.VMEM((CH,), jnp.int32),
        pltpu.VMEM((CH,), jnp.int32),
        pltpu.VMEM((CH,), jnp.float32),
    ],
)
def _sc_dist(px_hbm, py_hbm, pz_hbm, src_hbm, dst_hbm, d2_hbm,
             px_v, py_v, pz_v, src_v, dst_v, d2_v):
    wid = lax.axis_index("s") * 2 + lax.axis_index("c")
    pltpu.sync_copy(px_hbm, px_v)
    pltpu.sync_copy(py_hbm, py_v)
    pltpu.sync_copy(pz_hbm, pz_v)

    @pl.loop(0, ITERS)
    def _(i):
        cid = i * NW + wid

        @pl.when(cid < NCHUNK)
        def _():
            base = cid * CH
            pltpu.sync_copy(src_hbm.at[pl.ds(base, CH)], src_v)
            pltpu.sync_copy(dst_hbm.at[pl.ds(base, CH)], dst_v)
            for j in range(CH // 16):
                sl = pl.ds(j * 16, 16)
                si = src_v[sl]
                di = dst_v[sl]
                dx = plsc.load_gather(px_v, [si]) - plsc.load_gather(px_v, [di])
                dy = plsc.load_gather(py_v, [si]) - plsc.load_gather(py_v, [di])
                dz = plsc.load_gather(pz_v, [si]) - plsc.load_gather(pz_v, [di])
                d2_v[sl] = dx * dx + dy * dy + dz * dz
            pltpu.sync_copy(d2_v, d2_hbm.at[pl.ds(base, CH)])


@functools.partial(
    pl.kernel,
    out_type=jax.ShapeDtypeStruct((2, NPAD, C_FEAT), jnp.float32),
    mesh=_MESH,
    scratch_types=[
        pltpu.VMEM((CH,), jnp.int32),
        pltpu.VMEM((CH,), jnp.int32),
        pltpu.VMEM((CH, C_FEAT), jnp.float32),
        pltpu.VMEM((CH, C_FEAT), jnp.float32),
        pltpu.VMEM_SHARED((NPAD, C_FEAT), jnp.float32),
    ],
)
def _sc_edge(h_hbm, k_hbm, src_hbm, dst_hbm, out_hbm,
             src_v, dst_v, hrows_v, krows_v, acc_sh):
    cix = lax.axis_index("c")
    sid = lax.axis_index("s")
    wid = sid * 2 + cix

    @pl.loop(0, CH)
    def _(r):
        for c in range(8):
            krows_v[r, pl.ds(c * 16, 16)] = jnp.zeros((16,), jnp.float32)

    @pl.loop(0, STRIPE // CH)
    def _(j):
        pltpu.sync_copy(krows_v, acc_sh.at[pl.ds(sid * STRIPE + j * CH, CH)])

    plsc.subcore_barrier()

    @pl.loop(0, ITERS)
    def _(i):
        cid = i * NW + wid

        @pl.when(cid < NCHUNK)
        def _():
            base = cid * CH
            pltpu.sync_copy(src_hbm.at[pl.ds(base, CH)], src_v)
            pltpu.sync_copy(dst_hbm.at[pl.ds(base, CH)], dst_v)
            pltpu.sync_copy(h_hbm.at[src_v], hrows_v)
            pltpu.sync_copy(k_hbm.at[pl.ds(base, CH)], krows_v)

            @pl.loop(0, CH)
            def _(r):
                for c in range(8):
                    sl = pl.ds(c * 16, 16)
                    krows_v[r, sl] = krows_v[r, sl] * hrows_v[r, sl]

            pltpu.sync_copy(krows_v, acc_sh.at[dst_v], add=True)

    plsc.subcore_barrier()
    pltpu.sync_copy(acc_sh.at[pl.ds(sid * STRIPE, STRIPE)],
                    out_hbm.at[cix, pl.ds(sid * STRIPE, STRIPE)])


def _embed_body(x_ref, w_ref, o_ref):
    o_ref[...] = _dot(x_ref[...], w_ref[...])


def _basis_body(d2_ref, w1t_ref, b1_ref, w2t_ref, b2_ref, wk0_ref, wk1_ref,
                k0_ref, k1_ref):
    d2b = d2_ref[...]
    db = jnp.sqrt(d2b)
    d3b = d2b * db
    w1t = w1t_ref[...]
    w2t = w2t_ref[...]
    b1c = b1_ref[...]
    b2c = b2_ref[...]
    zrows = jnp.zeros((5, CH), jnp.float32)
    for j in range(8):
        p8 = jnp.concatenate(
            [db[j:j + 1, :], d2b[j:j + 1, :], d3b[j:j + 1, :], zrows], axis=0)
        t1 = jax.nn.gelu(_dot(w1t, p8) + b1c)
        t2 = jax.nn.gelu(_dot(w2t, t1) + b2c)
        sl = pl.ds(j * CH, CH)
        k0_ref[sl, :] = lax.dot_general(
            t2, wk0_ref[...], (((0,), (0,)), ((), ())),
            preferred_element_type=jnp.float32, precision=_HI)
        k1_ref[sl, :] = lax.dot_general(
            t2, wk1_ref[...], (((0,), (0,)), ((), ())),
            preferred_element_type=jnp.float32, precision=_HI)


def _node_body(p0_ref, p1_ref, h_ref, g_ref, b_ref, wm1_ref, bm1_ref,
               wm2_ref, bm2_ref, o_ref):
    agg = p0_ref[...] + p1_ref[...]
    mu = jnp.mean(agg, axis=1, keepdims=True)
    var = jnp.mean((agg - mu) ** 2, axis=1, keepdims=True)
    y = (agg - mu) / jnp.sqrt(var + 1e-5) * g_ref[...] + b_ref[...]
    z = jax.nn.gelu(_dot(y, wm1_ref[...]) + bm1_ref[...])
    o_ref[...] = h_ref[...] + _dot(z, wm2_ref[...]) + bm2_ref[...]


def _final_body(p0_ref, p1_ref, h_ref, bat_ref, g_ref, b_ref, wm1_ref,
                bm1_ref, wm2_ref, bm2_ref, wr0_ref, br0_ref, wr1_ref,
                br1_ref, o_ref):
    agg = p0_ref[...] + p1_ref[...]
    mu = jnp.mean(agg, axis=1, keepdims=True)
    var = jnp.mean((agg - mu) ** 2, axis=1, keepdims=True)
    y = (agg - mu) / jnp.sqrt(var + 1e-5) * g_ref[...] + b_ref[...]
    z = jax.nn.gelu(_dot(y, wm1_ref[...]) + bm1_ref[...])
    h1 = h_ref[...]
    h2 = h1 + _dot(z, wm2_ref[...]) + bm2_ref[...]
    tot = (jnp.sum(h1 * wr0_ref[...], axis=1, keepdims=True) + br0_ref[...]
           + jnp.sum(h2 * wr1_ref[...], axis=1, keepdims=True) + br1_ref[...])
    rows = tot.shape[0]
    gid = lax.broadcasted_iota(jnp.int32, (rows, NG), 1)
    onehot = (bat_ref[...] == gid).astype(jnp.float32)
    contrib = jnp.sum(onehot * tot, axis=0, keepdims=True)

    @pl.when(pl.program_id(0) == 0)
    def _():
        o_ref[...] = jnp.zeros_like(o_ref)

    o_ref[...] += contrib


def _full(shape):
    return pl.BlockSpec(shape, lambda i: tuple(0 for _ in shape))


def kernel(pos, x, batch, edge_index, W_embed, basis_W1, basis_b1, basis_W2,
           basis_b2, Wk0, ln_g0, ln_b0, Wm1_0, bm1_0, Wm2_0, bm2_0, Wr0, br0,
           Wk1, ln_g1, ln_b1, Wm1_1, bm1_1, Wm2_1, bm2_1, Wr1, br1):
    src = edge_index[0]
    dst = edge_index[1]

    RB = 1000
    h0 = pl.pallas_call(
        _embed_body,
        grid=(N // RB,),
        in_specs=[pl.BlockSpec((RB, C_FEAT), lambda i: (i, 0)),
                  _full((C_FEAT, C_FEAT))],
        out_specs=pl.BlockSpec((RB, C_FEAT), lambda i: (i, 0)),
        out_shape=jax.ShapeDtypeStruct((N, C_FEAT), jnp.float32),
    )(x, W_embed)

    d2 = _sc_dist(pos[:, 0], pos[:, 1], pos[:, 2], src, dst)

    D2R = NCHUNK + 4
    E2 = (D2R // 8) * 1024
    d2p = jnp.pad(d2.reshape(NCHUNK, CH), ((0, 4), (0, 0)))
    w1t = jnp.zeros((C_FEAT, 8), jnp.float32).at[:, :3].set(basis_W1.T)
    k0e, k1e = pl.pallas_call(
        _basis_body,
        grid=(D2R // 8,),
        in_specs=[pl.BlockSpec((8, CH), lambda i: (i, 0)),
                  _full((C_FEAT, 8)), _full((C_FEAT, 1)),
                  _full((C_FEAT, C_FEAT)), _full((C_FEAT, 1)),
                  _full((C_FEAT, C_FEAT)), _full((C_FEAT, C_FEAT))],
        out_specs=[pl.BlockSpec((1024, C_FEAT), lambda i: (i, 0)),
                   pl.BlockSpec((1024, C_FEAT), lambda i: (i, 0))],
        out_shape=[jax.ShapeDtypeStruct((E2, C_FEAT), jnp.float32),
                   jax.ShapeDtypeStruct((E2, C_FEAT), jnp.float32)],
    )(d2p, w1t, basis_b1.reshape(-1, 1), basis_W2.T,
      basis_b2.reshape(-1, 1), Wk0, Wk1)

    node_specs = [pl.BlockSpec((RB, C_FEAT), lambda i: (i, 0))] * 3 + [
        _full((1, C_FEAT)), _full((1, C_FEAT)),
        _full((C_FEAT, WIDEC)), _full((1, WIDEC)),
        _full((WIDEC, C_FEAT)), _full((1, C_FEAT))]

    part = _sc_edge(h0, k0e, src, dst)
    h1 = pl.pallas_call(
        _node_body,
        grid=(N // RB,),
        in_specs=node_specs,
        out_specs=pl.BlockSpec((RB, C_FEAT), lambda i: (i, 0)),
        out_shape=jax.ShapeDtypeStruct((N, C_FEAT), jnp.float32),
    )(part[0, :N], part[1, :N], h0, ln_g0.reshape(1, -1),
      ln_b0.reshape(1, -1), Wm1_0, bm1_0.reshape(1, -1), Wm2_0,
      bm2_0.reshape(1, -1))

    part = _sc_edge(h1, k1e, src, dst)
    out = pl.pallas_call(
        _final_body,
        grid=(N // RB,),
        in_specs=node_specs[:3] + [pl.BlockSpec((RB, 1), lambda i: (i, 0))]
        + node_specs[3:] + [_full((1, C_FEAT)), _full((1, 1)),
                            _full((1, C_FEAT)), _full((1, 1))],
        out_specs=_full((1, NG)),
        out_shape=jax.ShapeDtypeStruct((1, NG), jnp.float32),
    )(part[0, :N], part[1, :N], h1, batch.reshape(N, 1),
      ln_g1.reshape(1, -1), ln_b1.reshape(1, -1), Wm1_1,
      bm1_1.reshape(1, -1), Wm2_1, bm2_1.reshape(1, -1),
      Wr0.reshape(1, -1), br0.reshape(1, 1), Wr1.reshape(1, -1),
      br1.reshape(1, 1))
    return out.reshape(NG, 1)


# speedup vs baseline: 2.0856x; 2.0856x over previous
import dataclasses
import functools

import jax
import jax.numpy as jnp
from jax import lax
from jax.experimental import pallas as pl
from jax.experimental.pallas import tpu as pltpu
from jax.experimental.pallas import tpu_sc as plsc

N = 10000
E = 320000
C_FEAT = 128
WIDEC = 512
NG = 16

NW = 32
CH = 128
NCHUNK = E // CH
ITERS = -(-NCHUNK // NW)
CHE = 80
NCHE = E // CHE
ITE = NCHE // NW
NPAD = 10240
STRIPE = NPAD // 16

_MESH = plsc.VectorSubcoreMesh(core_axis_name="c", subcore_axis_name="s")
_HI = jax.lax.Precision.HIGHEST

_CP = pltpu.CompilerParams()
if "needs_layout_passes" in pltpu.CompilerParams.__dataclass_fields__:
    _CP = dataclasses.replace(_CP, needs_layout_passes=False)


def _dot(a, b):
    return jnp.dot(a, b, preferred_element_type=jnp.float32, precision=_HI)


@functools.partial(
    pl.kernel,
    out_type=jax.ShapeDtypeStruct((E,), jnp.float32),
    mesh=_MESH,
    compiler_params=_CP,
    scratch_types=[
        pltpu.VMEM((N,), jnp.float32),
        pltpu.VMEM((N,), jnp.float32),
        pltpu.VMEM((N,), jnp.float32),
        pltpu.VMEM((CH,), jnp.int32),
        pltpu.VMEM((CH,), jnp.int32),
        pltpu.VMEM((CH,), jnp.float32),
    ],
)
def _sc_dist(px_hbm, py_hbm, pz_hbm, src_hbm, dst_hbm, d2_hbm,
             px_v, py_v, pz_v, src_v, dst_v, d2_v):
    wid = lax.axis_index("s") * 2 + lax.axis_index("c")
    pltpu.sync_copy(px_hbm, px_v)
    pltpu.sync_copy(py_hbm, py_v)
    pltpu.sync_copy(pz_hbm, pz_v)

    @pl.loop(0, ITERS)
    def _(i):
        cid = i * NW + wid

        @pl.when(cid < NCHUNK)
        def _():
            base = cid * CH
            pltpu.sync_copy(src_hbm.at[pl.ds(base, CH)], src_v)
            pltpu.sync_copy(dst_hbm.at[pl.ds(base, CH)], dst_v)
            for j in range(CH // 16):
                sl = pl.ds(j * 16, 16)
                si = src_v[sl]
                di = dst_v[sl]
                dx = plsc.load_gather(px_v, [si]) - plsc.load_gather(px_v, [di])
                dy = plsc.load_gather(py_v, [si]) - plsc.load_gather(py_v, [di])
                dz = plsc.load_gather(pz_v, [si]) - plsc.load_gather(pz_v, [di])
                d2_v[sl] = dx * dx + dy * dy + dz * dz
            pltpu.sync_copy(d2_v, d2_hbm.at[pl.ds(base, CH)])


@functools.partial(
    pl.kernel,
    out_type=jax.ShapeDtypeStruct((2, NPAD, C_FEAT), jnp.float32),
    mesh=_MESH,
    scratch_types=[
        pltpu.VMEM((CHE,), jnp.int32), pltpu.VMEM((CHE,), jnp.int32),
        pltpu.VMEM((CHE,), jnp.int32), pltpu.VMEM((CHE,), jnp.int32),
        pltpu.VMEM((CHE, C_FEAT), jnp.float32),
        pltpu.VMEM((CHE, C_FEAT), jnp.float32),
        pltpu.VMEM((CHE, C_FEAT), jnp.float32),
        pltpu.VMEM((CHE, C_FEAT), jnp.float32),
        pltpu.VMEM_SHARED((NPAD, C_FEAT), jnp.float32),
        pltpu.SemaphoreType.DMA, pltpu.SemaphoreType.DMA,
        pltpu.SemaphoreType.DMA, pltpu.SemaphoreType.DMA,
    ],
)
def _sc_edge(h_hbm, k_hbm, src_hbm, dst_hbm, out_hbm,
             src0, src1, dst0, dst1, hb0, hb1, kb0, kb1, acc_sh,
             sg0, sg1, sk0, sk1):
    cix = lax.axis_index("c")
    sid = lax.axis_index("s")
    wid = sid * 2 + cix
    srcv = [src0, src1]
    dstv = [dst0, dst1]
    hv = [hb0, hb1]
    kv = [kb0, kb1]
    sg = [sg0, sg1]
    sk = [sk0, sk1]

    @pl.loop(0, CHE)
    def _(r):
        for c in range(8):
            kb0[r, pl.ds(c * 16, 16)] = jnp.zeros((16,), jnp.float32)

    @pl.loop(0, STRIPE // CHE)
    def _(j):
        pltpu.sync_copy(kb0, acc_sh.at[pl.ds(sid * STRIPE + j * CHE, CHE)])

    plsc.subcore_barrier()

    @pl.loop(0, (ITE + 1) // 2)
    def _(t):
        for b in range(2):
            ju = t * 2 + b

            @pl.when(ju < ITE)
            def _():
                base = (ju * NW + wid) * CHE
                pltpu.sync_copy(src_hbm.at[pl.ds(base, CHE)], srcv[b])
                pltpu.sync_copy(dst_hbm.at[pl.ds(base, CHE)], dstv[b])
                pltpu.make_async_copy(h_hbm.at[srcv[b]], hv[b], sg[b]).start()
                pltpu.make_async_copy(
                    k_hbm.at[pl.ds(base, CHE)], kv[b], sk[b]).start()

            jp = ju - 1
            p = 1 - b

            @pl.when(jp >= 0)
            def _():
                pltpu.make_async_copy(h_hbm.at[srcv[p]], hv[p], sg[p]).wait()
                pltpu.make_async_copy(
                    k_hbm.at[pl.ds(0, CHE)], kv[p], sk[p]).wait()

                @pl.loop(0, CHE)
                def _(r):
                    for c in range(8):
                        sl = pl.ds(c * 16, 16)
                        kv[p][r, sl] = kv[p][r, sl] * hv[p][r, sl]

                pltpu.sync_copy(kv[p], acc_sh.at[dstv[p]], add=True)

    plsc.subcore_barrier()
    pltpu.sync_copy(acc_sh.at[pl.ds(sid * STRIPE, STRIPE)],
                    out_hbm.at[cix, pl.ds(sid * STRIPE, STRIPE)])


def _embed_body(x_ref, w_ref, o_ref):
    o_ref[...] = _dot(x_ref[...], w_ref[...])


def _basis_body(d2_ref, w1_ref, b1_ref, w2_ref, b2_ref, wk0_ref, wk1_ref,
                k0_ref, k1_ref):
    d2t = jnp.transpose(d2_ref[...])
    d2 = jnp.concatenate([d2t[:, j:j + 1] for j in range(8)], axis=0)
    d = jnp.sqrt(d2)
    d3 = d2 * d
    w1 = w1_ref[...]
    t = d * w1[0:1, :] + d2 * w1[1:2, :] + d3 * w1[2:3, :] + b1_ref[...]
    t = jax.nn.gelu(t)
    t = jax.nn.gelu(_dot(t, w2_ref[...]) + b2_ref[...])
    k0_ref[...] = _dot(t, wk0_ref[...])
    k1_ref[...] = _dot(t, wk1_ref[...])


def _node_body(p0_ref, p1_ref, h_ref, g_ref, b_ref, wm1_ref, bm1_ref,
               wm2_ref, bm2_ref, o_ref):
    agg = p0_ref[...] + p1_ref[...]
    mu = jnp.mean(agg, axis=1, keepdims=True)
    var = jnp.mean((agg - mu) ** 2, axis=1, keepdims=True)
    y = (agg - mu) / jnp.sqrt(var + 1e-5) * g_ref[...] + b_ref[...]
    z = jax.nn.gelu(_dot(y, wm1_ref[...]) + bm1_ref[...])
    o_ref[...] = h_ref[...] + _dot(z, wm2_ref[...]) + bm2_ref[...]


def _final_body(p0_ref, p1_ref, h_ref, bat_ref, g_ref, b_ref, wm1_ref,
                bm1_ref, wm2_ref, bm2_ref, wr0_ref, br0_ref, wr1_ref,
                br1_ref, o_ref):
    agg = p0_ref[...] + p1_ref[...]
    mu = jnp.mean(agg, axis=1, keepdims=True)
    var = jnp.mean((agg - mu) ** 2, axis=1, keepdims=True)
    y = (agg - mu) / jnp.sqrt(var + 1e-5) * g_ref[...] + b_ref[...]
    z = jax.nn.gelu(_dot(y, wm1_ref[...]) + bm1_ref[...])
    h1 = h_ref[...]
    h2 = h1 + _dot(z, wm2_ref[...]) + bm2_ref[...]
    tot = (jnp.sum(h1 * wr0_ref[...], axis=1, keepdims=True) + br0_ref[...]
           + jnp.sum(h2 * wr1_ref[...], axis=1, keepdims=True) + br1_ref[...])
    rows = tot.shape[0]
    gid = lax.broadcasted_iota(jnp.int32, (rows, NG), 1)
    onehot = (bat_ref[...] == gid).astype(jnp.float32)
    contrib = jnp.sum(onehot * tot, axis=0, keepdims=True)

    @pl.when(pl.program_id(0) == 0)
    def _():
        o_ref[...] = jnp.zeros_like(o_ref)

    o_ref[...] += contrib


def _full(shape):
    return pl.BlockSpec(shape, lambda i: tuple(0 for _ in shape))


def kernel(pos, x, batch, edge_index, W_embed, basis_W1, basis_b1, basis_W2,
           basis_b2, Wk0, ln_g0, ln_b0, Wm1_0, bm1_0, Wm2_0, bm2_0, Wr0, br0,
           Wk1, ln_g1, ln_b1, Wm1_1, bm1_1, Wm2_1, bm2_1, Wr1, br1):
    src = edge_index[0]
    dst = edge_index[1]

    RB = 1000
    h0 = pl.pallas_call(
        _embed_body,
        grid=(N // RB,),
        in_specs=[pl.BlockSpec((RB, C_FEAT), lambda i: (i, 0)),
                  _full((C_FEAT, C_FEAT))],
        out_specs=pl.BlockSpec((RB, C_FEAT), lambda i: (i, 0)),
        out_shape=jax.ShapeDtypeStruct((N, C_FEAT), jnp.float32),
    )(x, W_embed)

    d2 = _sc_dist(pos[:, 0], pos[:, 1], pos[:, 2], src, dst)

    D2R = NCHUNK + 4
    E2 = (D2R // 8) * 1024
    d2p = jnp.pad(d2.reshape(NCHUNK, CH), ((0, 4), (0, 0)))
    w1p = jnp.zeros((8, C_FEAT), jnp.float32).at[:3].set(basis_W1)
    k0e, k1e = pl.pallas_call(
        _basis_body,
        grid=(D2R // 8,),
        in_specs=[pl.BlockSpec((8, CH), lambda i: (i, 0)),
                  _full((8, C_FEAT)), _full((1, C_FEAT)),
                  _full((C_FEAT, C_FEAT)), _full((1, C_FEAT)),
                  _full((C_FEAT, C_FEAT)), _full((C_FEAT, C_FEAT))],
        out_specs=[pl.BlockSpec((1024, C_FEAT), lambda i: (i, 0)),
                   pl.BlockSpec((1024, C_FEAT), lambda i: (i, 0))],
        out_shape=[jax.ShapeDtypeStruct((E2, C_FEAT), jnp.float32),
                   jax.ShapeDtypeStruct((E2, C_FEAT), jnp.float32)],
    )(d2p, w1p, basis_b1.reshape(1, -1), basis_W2,
      basis_b2.reshape(1, -1), Wk0, Wk1)

    node_specs = [pl.BlockSpec((RB, C_FEAT), lambda i: (i, 0))] * 3 + [
        _full((1, C_FEAT)), _full((1, C_FEAT)),
        _full((C_FEAT, WIDEC)), _full((1, WIDEC)),
        _full((WIDEC, C_FEAT)), _full((1, C_FEAT))]

    part = _sc_edge(h0, k0e, src, dst)
    h1 = pl.pallas_call(
        _node_body,
        grid=(N // RB,),
        in_specs=node_specs,
        out_specs=pl.BlockSpec((RB, C_FEAT), lambda i: (i, 0)),
        out_shape=jax.ShapeDtypeStruct((N, C_FEAT), jnp.float32),
    )(part[0, :N], part[1, :N], h0, ln_g0.reshape(1, -1),
      ln_b0.reshape(1, -1), Wm1_0, bm1_0.reshape(1, -1), Wm2_0,
      bm2_0.reshape(1, -1))

    part = _sc_edge(h1, k1e, src, dst)
    out = pl.pallas_call(
        _final_body,
        grid=(N // RB,),
        in_specs=node_specs[:3] + [pl.BlockSpec((RB, 1), lambda i: (i, 0))]
        + node_specs[3:] + [_full((1, C_FEAT)), _full((1, 1)),
                            _full((1, C_FEAT)), _full((1, 1))],
        out_specs=_full((1, NG)),
        out_shape=jax.ShapeDtypeStruct((1, NG), jnp.float32),
    )(part[0, :N], part[1, :N], h1, batch.reshape(N, 1),
      ln_g1.reshape(1, -1), ln_b1.reshape(1, -1), Wm1_1,
      bm1_1.reshape(1, -1), Wm2_1, bm2_1.reshape(1, -1),
      Wr0.reshape(1, -1), br0.reshape(1, 1), Wr1.reshape(1, -1),
      br1.reshape(1, 1))
    return out.reshape(NG, 1)
